# baseline (speedup 1.0000x reference)
import jax
import jax.numpy as jnp
from jax import lax
from jax.experimental import pallas as pl
from jax.experimental.pallas import tpu as pltpu
from jax.experimental.pallas import tpu_sc as plsc

N = 10000
E = 320000
D = 128
DW = 144
H = 8
DH = 16
NSUB = 16
EPT = E // NSUB
K = 40
NCH = EPT // K
NBUF = 4
SBC = 25
NSB = NCH // SBC
RPT = 624
RPT_LAST = N - RPT * (NSUB - 1)
BN = 2000


def _proj_body(x_ref, w_ref, ac_ref, h_ref, a_ref):
    h = jnp.dot(x_ref[...], w_ref[...], preferred_element_type=jnp.float32)
    h_ref[...] = jnp.concatenate(
        [h, jnp.zeros((h.shape[0], DW - D), jnp.float32)], axis=1)
    a_ref[...] = jnp.dot(h, ac_ref[...], preferred_element_type=jnp.float32)


_proj_call = pl.pallas_call(
    _proj_body,
    grid=(10,),
    in_specs=[pl.BlockSpec((1000, D), lambda i: (i, 0)),
              pl.BlockSpec((D, D), lambda i: (0, 0)),
              pl.BlockSpec((D, 2 * DH), lambda i: (0, 0))],
    out_specs=[pl.BlockSpec((1000, DW), lambda i: (i, 0)),
               pl.BlockSpec((1000, 2 * DH), lambda i: (i, 0))],
    out_shape=[jax.ShapeDtypeStruct((N, DW), jnp.float32),
               jax.ShapeDtypeStruct((N, 2 * DH), jnp.float32)],
)


def _gat_sc_body(h_hbm, s_hbm, d_hbm, src_hbm, dst_hbm, nd_hbm,
                 nd_sp, sblk, dblk, sgs, dgs, hrs,
                 g0, g1, g2, g3, s0, s1, s2, s3, sb_sem):
    s = lax.axis_index("s")
    GS = [g0, g1, g2, g3]
    SS = [s0, s1, s2, s3]
    base_c = s * NCH

    def _zb(j, carry):
        for t in range(DW // DH):
            hrs[0, j, pl.ds(t * DH, DH)] = jnp.zeros((DH,), jnp.float32)
        return carry
    lax.fori_loop(0, K, _zb, 0)

    @pl.when(s < NSUB - 1)
    def _():
        for r in range(RPT // DH):
            pltpu.sync_copy(hrs.at[0, pl.ds(0, DH)],
                            nd_sp.at[pl.ds(s * RPT + r * DH, DH)])

    @pl.when(s == NSUB - 1)
    def _():
        base_r = RPT * (NSUB - 1)
        for r in range(RPT_LAST // DH):
            pltpu.sync_copy(hrs.at[0, pl.ds(0, DH)],
                            nd_sp.at[pl.ds(base_r + r * DH, DH)])
    plsc.subcore_barrier()

    def _sv(c):
        return sblk.at[(c // SBC) % 2, c % SBC]

    def _dv(c):
        return dblk.at[(c // SBC) % 2, c % SBC]

    def _issue(c, b):
        pltpu.async_copy(s_hbm.at[_sv(c)], sgs.at[b], GS[b])
        pltpu.async_copy(d_hbm.at[_dv(c)], dgs.at[b], GS[b])
        pltpu.async_copy(h_hbm.at[_sv(c)], hrs.at[b], GS[b])

    def _wait_gather(c, b):
        pltpu.make_async_copy(s_hbm.at[_sv(c)], sgs.at[b], GS[b]).wait()
        pltpu.make_async_copy(d_hbm.at[_dv(c)], dgs.at[b], GS[b]).wait()
        pltpu.make_async_copy(h_hbm.at[_sv(c)], hrs.at[b], GS[b]).wait()

    def _wait_scatter(c, b):
        pltpu.make_async_copy(hrs.at[b], nd_sp.at[_dv(c)], SS[b]).wait()

    pltpu.sync_copy(src_hbm.at[pl.ds(base_c, SBC)], sblk.at[0])
    pltpu.sync_copy(dst_hbm.at[pl.ds(base_c, SBC)], dblk.at[0])
    for b in range(NBUF - 1):
        _issue(b, b)
    pltpu.async_copy(src_hbm.at[pl.ds(base_c + SBC, SBC)], sblk.at[1], sb_sem)
    pltpu.async_copy(dst_hbm.at[pl.ds(base_c + SBC, SBC)], dblk.at[1], sb_sem)

    def _outer(g, carry):
        for b in range(NBUF):
            c = g * NBUF + b
            _wait_gather(c, b)

            @plsc.parallel_loop(0, K, unroll=4)
            def _ebody(j):
                ev = sgs[b, j, :] + dgs[b, j, :]
                ev = jnp.maximum(ev, 0.2 * ev)
                wv = jnp.exp(ev)
                hrs[b, j, pl.ds(D, DH)] = wv
                for t in range(H):
                    sc = wv.at[jnp.full((DH,), t, jnp.int32)].get(
                        mode="promise_in_bounds")
                    hrs[b, j, pl.ds(t * DH, DH)] = (
                        hrs[b, j, pl.ds(t * DH, DH)] * sc)

            pltpu.async_copy(hrs.at[b], nd_sp.at[_dv(c)], SS[b], add=True)

            bn = (b + NBUF - 1) % NBUF
            cn = c + NBUF - 1

            @pl.when(cn < NCH)
            def _():
                @pl.when(c >= 1)
                def _():
                    _wait_scatter(c - 1, bn)

                @pl.when((cn % SBC == 0) & (cn // SBC > 0))
                def _():
                    sbn = cn // SBC
                    pltpu.make_async_copy(
                        src_hbm.at[pl.ds(base_c + sbn * SBC, SBC)],
                        sblk.at[sbn % 2], sb_sem).wait()
                    pltpu.make_async_copy(
                        dst_hbm.at[pl.ds(base_c + sbn * SBC, SBC)],
                        dblk.at[sbn % 2], sb_sem).wait()

                _issue(cn, bn)

            @pl.when((c % SBC == 0) & (c > 0) & (c // SBC + 1 < NSB))
            def _():
                sbp = c // SBC + 1
                pltpu.async_copy(
                    src_hbm.at[pl.ds(base_c + sbp * SBC, SBC)],
                    sblk.at[sbp % 2], sb_sem)
                pltpu.async_copy(
                    dst_hbm.at[pl.ds(base_c + sbp * SBC, SBC)],
                    dblk.at[sbp % 2], sb_sem)
        return carry
    lax.fori_loop(0, NCH // NBUF, _outer, 0)

    for b in range(NBUF):
        _wait_scatter(NCH - NBUF + b, b)
    plsc.subcore_barrier()

    @pl.when(s < NSUB - 1)
    def _():
        pltpu.sync_copy(nd_sp.at[pl.ds(s * RPT, RPT)],
                        nd_hbm.at[pl.ds(s * RPT, RPT)])

    @pl.when(s == NSUB - 1)
    def _():
        base_r = RPT * (NSUB - 1)
        pltpu.sync_copy(nd_sp.at[pl.ds(base_r, RPT_LAST)],
                        nd_hbm.at[pl.ds(base_r, RPT_LAST)])


_sc_call = pl.kernel(
    _gat_sc_body,
    out_type=pltpu.HBM((N, DW), jnp.float32),
    mesh=plsc.VectorSubcoreMesh(core_axis_name="c", subcore_axis_name="s",
                                num_cores=1),
    scratch_types=(
        [pltpu.VMEM_SHARED((N, DW), jnp.float32),
         pltpu.VMEM((2, SBC, K), jnp.int32),
         pltpu.VMEM((2, SBC, K), jnp.int32),
         pltpu.VMEM((NBUF, K, DH), jnp.float32),
         pltpu.VMEM((NBUF, K, DH), jnp.float32),
         pltpu.VMEM((NBUF, K, DW), jnp.float32)]
        + [pltpu.SemaphoreType.DMA for _ in range(9)]
    ),
    compiler_params=pltpu.CompilerParams(use_tc_tiling_on_sc=False,
                                         needs_layout_passes=False),
)


def _combine_body(nd_ref, b_ref, o_ref):
    num = nd_ref[:, :D]
    den = nd_ref[:, D:]
    cc = lax.broadcasted_iota(jnp.int32, (DH, D), 0)
    jj = lax.broadcasted_iota(jnp.int32, (DH, D), 1)
    expand = (jj // DH == cc).astype(jnp.float32)
    den_full = jnp.dot(den, expand, preferred_element_type=jnp.float32)
    z = num / (den_full + 1e-16) + b_ref[...][None, :]
    o_ref[...] = jnp.where(z > 0, z, jnp.exp(jnp.minimum(z, 0.0)) - 1.0)


_combine_call = pl.pallas_call(
    _combine_body,
    grid=(N // BN,),
    in_specs=[pl.BlockSpec((BN, DW), lambda i: (i, 0)),
              pl.BlockSpec((D,), lambda i: (0,))],
    out_specs=pl.BlockSpec((BN, D), lambda i: (i, 0)),
    out_shape=jax.ShapeDtypeStruct((N, D), jnp.float32),
)


def kernel(x, edge_index, W, att_src, att_dst, bias):
    src = edge_index[0].reshape(E // K, K)
    dst = edge_index[1].reshape(E // K, K)
    hh = jnp.arange(D, dtype=jnp.int32) // DH
    dd = jnp.arange(D, dtype=jnp.int32) % DH
    cm = jnp.arange(2 * H, dtype=jnp.int32) % H
    vs = att_src[0][hh, dd]
    vd = att_dst[0][hh, dd]
    m = (hh[:, None] == cm[None, :]).astype(jnp.float32)
    acat = jnp.concatenate([vs[:, None] * m, vd[:, None] * m], axis=1)

    h144, a = _proj_call(x, W, acat)
    nd = _sc_call(h144, a[:, :2 * H], a[:, 2 * H:], src, dst)
    out = _combine_call(nd, bias)
    return out


# speedup vs baseline: 1.0015x; 1.0015x over previous
import jax
import jax.numpy as jnp
from jax import lax
from jax.experimental import pallas as pl
from jax.experimental.pallas import tpu as pltpu
from jax.experimental.pallas import tpu_sc as plsc

N = 10000
E = 320000
D = 128
DW = 144
H = 8
DH = 16
NSUB = 16
EPT = E // NSUB
K = 40
NCH = EPT // K
NBUF = 4
SBC = 25
NSB = NCH // SBC
RPT = 624
RPT_LAST = N - RPT * (NSUB - 1)
BN = 2000


def _proj_body(x_ref, w_ref, ac_ref, h_ref, a_ref):
    h = jnp.dot(x_ref[...], w_ref[...], preferred_element_type=jnp.float32)
    h_ref[...] = jnp.concatenate(
        [h, jnp.zeros((h.shape[0], DW - D), jnp.float32)], axis=1)
    a_ref[...] = jnp.dot(h, ac_ref[...], preferred_element_type=jnp.float32)


_proj_call = pl.pallas_call(
    _proj_body,
    grid=(10,),
    in_specs=[pl.BlockSpec((1000, D), lambda i: (i, 0)),
              pl.BlockSpec((D, D), lambda i: (0, 0)),
              pl.BlockSpec((D, 2 * DH), lambda i: (0, 0))],
    out_specs=[pl.BlockSpec((1000, DW), lambda i: (i, 0)),
               pl.BlockSpec((1000, 2 * DH), lambda i: (i, 0))],
    out_shape=[jax.ShapeDtypeStruct((N, DW), jnp.float32),
               jax.ShapeDtypeStruct((N, 2 * DH), jnp.float32)],
)


def _gat_sc_body(h_hbm, s_hbm, d_hbm, src_hbm, dst_hbm, nd_hbm,
                 nd_sp, sblk, dblk, sgs, dgs, hrs,
                 g0, g1, g2, g3, s0, s1, s2, s3, sb_sem):
    s = lax.axis_index("s")
    GS = [g0, g1, g2, g3]
    SS = [s0, s1, s2, s3]
    base_c = s * NCH

    def _zb(j, carry):
        for t in range(DW // DH):
            hrs[0, j, pl.ds(t * DH, DH)] = jnp.zeros((DH,), jnp.float32)
        return carry
    lax.fori_loop(0, K, _zb, 0)

    @pl.when(s < NSUB - 1)
    def _():
        for r in range(RPT // DH):
            pltpu.sync_copy(hrs.at[0, pl.ds(0, DH)],
                            nd_sp.at[pl.ds(s * RPT + r * DH, DH)])

    @pl.when(s == NSUB - 1)
    def _():
        base_r = RPT * (NSUB - 1)
        for r in range(RPT_LAST // DH):
            pltpu.sync_copy(hrs.at[0, pl.ds(0, DH)],
                            nd_sp.at[pl.ds(base_r + r * DH, DH)])
    plsc.subcore_barrier()

    def _sv(c):
        return sblk.at[(c // SBC) % 2, c % SBC]

    def _dv(c):
        return dblk.at[(c // SBC) % 2, c % SBC]

    def _issue(c, b):
        pltpu.async_copy(s_hbm.at[_sv(c)], sgs.at[b], GS[b])
        pltpu.async_copy(d_hbm.at[_dv(c)], dgs.at[b], GS[b])
        pltpu.async_copy(h_hbm.at[_sv(c)], hrs.at[b], GS[b])

    def _wait_gather(c, b):
        pltpu.make_async_copy(s_hbm.at[_sv(c)], sgs.at[b], GS[b]).wait()
        pltpu.make_async_copy(d_hbm.at[_dv(c)], dgs.at[b], GS[b]).wait()
        pltpu.make_async_copy(h_hbm.at[_sv(c)], hrs.at[b], GS[b]).wait()

    def _wait_scatter(c, b):
        pltpu.make_async_copy(hrs.at[b], nd_sp.at[_dv(c)], SS[b]).wait()

    pltpu.sync_copy(src_hbm.at[pl.ds(base_c, SBC)], sblk.at[0])
    pltpu.sync_copy(dst_hbm.at[pl.ds(base_c, SBC)], dblk.at[0])
    for b in range(NBUF - 1):
        _issue(b, b)
    pltpu.async_copy(src_hbm.at[pl.ds(base_c + SBC, SBC)], sblk.at[1], sb_sem)
    pltpu.async_copy(dst_hbm.at[pl.ds(base_c + SBC, SBC)], dblk.at[1], sb_sem)

    def _outer(g, carry):
        for b in range(NBUF):
            c = g * NBUF + b
            _wait_gather(c, b)

            @plsc.parallel_loop(0, K, unroll=2)
            def _ebody(j):
                ev = sgs[b, j, :] + dgs[b, j, :]
                ev = jnp.maximum(ev, 0.2 * ev)
                wv = jnp.exp(ev)
                hrs[b, j, pl.ds(D, DH)] = wv
                for t in range(H):
                    sc = wv.at[jnp.full((DH,), t, jnp.int32)].get(
                        mode="promise_in_bounds")
                    hrs[b, j, pl.ds(t * DH, DH)] = (
                        hrs[b, j, pl.ds(t * DH, DH)] * sc)

            pltpu.async_copy(hrs.at[b], nd_sp.at[_dv(c)], SS[b], add=True)

            bn = (b + NBUF - 1) % NBUF
            cn = c + NBUF - 1

            @pl.when(cn < NCH)
            def _():
                @pl.when(c >= 1)
                def _():
                    _wait_scatter(c - 1, bn)

                @pl.when((cn % SBC == 0) & (cn // SBC > 0))
                def _():
                    sbn = cn // SBC
                    pltpu.make_async_copy(
                        src_hbm.at[pl.ds(base_c + sbn * SBC, SBC)],
                        sblk.at[sbn % 2], sb_sem).wait()
                    pltpu.make_async_copy(
                        dst_hbm.at[pl.ds(base_c + sbn * SBC, SBC)],
                        dblk.at[sbn % 2], sb_sem).wait()

                _issue(cn, bn)

            @pl.when((c % SBC == 0) & (c > 0) & (c // SBC + 1 < NSB))
            def _():
                sbp = c // SBC + 1
                pltpu.async_copy(
                    src_hbm.at[pl.ds(base_c + sbp * SBC, SBC)],
                    sblk.at[sbp % 2], sb_sem)
                pltpu.async_copy(
                    dst_hbm.at[pl.ds(base_c + sbp * SBC, SBC)],
                    dblk.at[sbp % 2], sb_sem)
        return carry
    lax.fori_loop(0, NCH // NBUF, _outer, 0)

    for b in range(NBUF):
        _wait_scatter(NCH - NBUF + b, b)
    plsc.subcore_barrier()

    @pl.when(s < NSUB - 1)
    def _():
        pltpu.sync_copy(nd_sp.at[pl.ds(s * RPT, RPT)],
                        nd_hbm.at[pl.ds(s * RPT, RPT)])

    @pl.when(s == NSUB - 1)
    def _():
        base_r = RPT * (NSUB - 1)
        pltpu.sync_copy(nd_sp.at[pl.ds(base_r, RPT_LAST)],
                        nd_hbm.at[pl.ds(base_r, RPT_LAST)])


_sc_call = pl.kernel(
    _gat_sc_body,
    out_type=pltpu.HBM((N, DW), jnp.float32),
    mesh=plsc.VectorSubcoreMesh(core_axis_name="c", subcore_axis_name="s",
                                num_cores=1),
    scratch_types=(
        [pltpu.VMEM_SHARED((N, DW), jnp.float32),
         pltpu.VMEM((2, SBC, K), jnp.int32),
         pltpu.VMEM((2, SBC, K), jnp.int32),
         pltpu.VMEM((NBUF, K, DH), jnp.float32),
         pltpu.VMEM((NBUF, K, DH), jnp.float32),
         pltpu.VMEM((NBUF, K, DW), jnp.float32)]
        + [pltpu.SemaphoreType.DMA for _ in range(9)]
    ),
    compiler_params=pltpu.CompilerParams(use_tc_tiling_on_sc=False,
                                         needs_layout_passes=False),
)


def _combine_body(nd_ref, b_ref, o_ref):
    num = nd_ref[:, :D]
    den = nd_ref[:, D:]
    cc = lax.broadcasted_iota(jnp.int32, (DH, D), 0)
    jj = lax.broadcasted_iota(jnp.int32, (DH, D), 1)
    expand = (jj // DH == cc).astype(jnp.float32)
    den_full = jnp.dot(den, expand, preferred_element_type=jnp.float32)
    z = num / (den_full + 1e-16) + b_ref[...][None, :]
    o_ref[...] = jnp.where(z > 0, z, jnp.exp(jnp.minimum(z, 0.0)) - 1.0)


_combine_call = pl.pallas_call(
    _combine_body,
    grid=(N // BN,),
    in_specs=[pl.BlockSpec((BN, DW), lambda i: (i, 0)),
              pl.BlockSpec((D,), lambda i: (0,))],
    out_specs=pl.BlockSpec((BN, D), lambda i: (i, 0)),
    out_shape=jax.ShapeDtypeStruct((N, D), jnp.float32),
)


def kernel(x, edge_index, W, att_src, att_dst, bias):
    src = edge_index[0].reshape(E // K, K)
    dst = edge_index[1].reshape(E // K, K)
    hh = jnp.arange(D, dtype=jnp.int32) // DH
    dd = jnp.arange(D, dtype=jnp.int32) % DH
    cm = jnp.arange(2 * H, dtype=jnp.int32) % H
    vs = att_src[0][hh, dd]
    vd = att_dst[0][hh, dd]
    m = (hh[:, None] == cm[None, :]).astype(jnp.float32)
    acat = jnp.concatenate([vs[:, None] * m, vd[:, None] * m], axis=1)

    h144, a = _proj_call(x, W, acat)
    nd = _sc_call(h144, a[:, :2 * H], a[:, 2 * H:], src, dst)
    out = _combine_call(nd, bias)
    return out
